# Initial kernel scaffold; baseline (speedup 1.0000x reference)
import jax
import jax.numpy as jnp
from jax import lax
from jax.experimental import pallas as pl
from jax.experimental.pallas import tpu as pltpu
from jax.experimental.pallas import tpu_sc as plsc

VOCAB = 1000000
DIM = 64
B = 16384
NNEG = 20

NC = 2
NS = 16
L = 16
NW = NC * NS
BPW = B // NW
CH = 32
NCH = BPW // CH
CN_PER_CHUNK = CH * NNEG
CN_DMA = 128
N_CN_DMAS = CN_PER_CHUNK // CN_DMA


def _sc_scores_body(t_tab, c_tab, t_idx, cp_idx, cn_idx, out,
                    idx_t, idx_cp, idx_cn, t_rows, cp_rows, cn_rows,
                    sp_out, sn_out, sem):
    wid = lax.axis_index("s") * NC + lax.axis_index("c")
    base = wid * BPW

    pltpu.sync_copy(t_idx.at[pl.ds(base, BPW)], idx_t)
    pltpu.sync_copy(cp_idx.at[pl.ds(base, BPW)], idx_cp)
    pltpu.sync_copy(cn_idx.at[pl.ds(base * NNEG, BPW * NNEG)], idx_cn)

    lane = lax.iota(jnp.int32, (L,))

    def do_chunk(c):
        cps = []
        cps.append(pltpu.async_copy(
            t_tab.at[idx_t.at[pl.ds(c * CH, CH)]], t_rows, sem))
        cps.append(pltpu.async_copy(
            c_tab.at[idx_cp.at[pl.ds(c * CH, CH)]], cp_rows, sem))
        for j in range(N_CN_DMAS):
            cps.append(pltpu.async_copy(
                c_tab.at[idx_cn.at[pl.ds(c * CN_PER_CHUNK + j * CN_DMA, CN_DMA)]],
                cn_rows.at[pl.ds(j * CN_DMA, CN_DMA)], sem))
        for h in cps:
            h.wait()

        for g in range(CH // L):
            b0 = g * L
            row_t = b0 + lane
            row_cn0 = (b0 + lane) * NNEG

            def dot_step(d, carry):
                s_p, s_n = carry
                dv = jnp.full((L,), d, jnp.int32)
                td = plsc.load_gather(t_rows, [row_t, dv])
                cpd = plsc.load_gather(cp_rows, [row_t, dv])
                cs = plsc.load_gather(cn_rows, [row_cn0, dv])
                for n in range(1, NNEG):
                    cs = cs + plsc.load_gather(cn_rows, [row_cn0 + n, dv])
                return s_p + td * cpd, s_n + td * cs

            zero = jnp.zeros((L,), jnp.float32)
            s_p, s_n = lax.fori_loop(0, DIM, dot_step, (zero, zero))
            sp_out[pl.ds(c * CH + b0, L)] = -s_p
            sn_out[pl.ds(c * CH + b0, L)] = s_n

    def loop_body(c, _):
        do_chunk(c)
        return 0

    lax.fori_loop(0, NCH, loop_body, 0)

    pltpu.sync_copy(sp_out, out.at[pl.ds(base, BPW)])
    pltpu.sync_copy(sn_out, out.at[pl.ds(B + base, BPW)])


def _sc_scores(t_tab, c_tab, t_idx, cp_idx, cn_idx):
    mesh = plsc.VectorSubcoreMesh(core_axis_name="c", subcore_axis_name="s")
    return pl.kernel(
        _sc_scores_body,
        out_type=jax.ShapeDtypeStruct((2 * B,), jnp.float32),
        mesh=mesh,
        scratch_types=[
            pltpu.VMEM((BPW,), jnp.int32),
            pltpu.VMEM((BPW,), jnp.int32),
            pltpu.VMEM((BPW * NNEG,), jnp.int32),
            pltpu.VMEM((CH, DIM), jnp.float32),
            pltpu.VMEM((CH, DIM), jnp.float32),
            pltpu.VMEM((CN_PER_CHUNK, DIM), jnp.float32),
            pltpu.VMEM((BPW,), jnp.float32),
            pltpu.VMEM((BPW,), jnp.float32),
            pltpu.SemaphoreType.DMA,
        ],
    )(t_tab, c_tab, t_idx, cp_idx, cn_idx)


def _loss_body(s_ref, o_ref):
    x = s_ref[...]
    z = jnp.minimum(x, 0.0) - jnp.log(1.0 + jnp.exp(-jnp.abs(x)))
    o_ref[0, 0] = jnp.sum(z) * (1.0 / B)


def _tc_loss(scores):
    out = pl.pallas_call(
        _loss_body,
        out_shape=jax.ShapeDtypeStruct((1, 1), jnp.float32),
        out_specs=pl.BlockSpec(memory_space=pltpu.SMEM),
    )(scores.reshape(128, 2 * B // 128))
    return out[0, 0]


@jax.jit
def kernel(t_vocab_embs, c_vocab_embs, t, cp, cn):
    t_i = t.astype(jnp.int32)
    cp_i = cp.astype(jnp.int32)
    cn_i = cn.astype(jnp.int32).reshape(B * NNEG)
    scores = _sc_scores(t_vocab_embs, c_vocab_embs, t_i, cp_i, cn_i)
    return _tc_loss(scores)


# speedup vs baseline: 4.3559x; 4.3559x over previous
import jax
import jax.numpy as jnp
from jax import lax
from jax.experimental import pallas as pl
from jax.experimental.pallas import tpu as pltpu
from jax.experimental.pallas import tpu_sc as plsc

VOCAB = 1000000
DIM = 64
B = 16384
NNEG = 20

NC = 2
NS = 16
L = 16
NW = NC * NS
BPW = B // NW
CH = 32
NCH = BPW // CH
CN_PER_CHUNK = CH * NNEG
CN_DMA = 128
N_CN_DMAS = CN_PER_CHUNK // CN_DMA


def _sc_scores_body(t_tab, c_tab, t_idx, cp_idx, cn_idx, out,
                    idx_t, idx_cp, idx_cn, t_rows, cp_rows, cn_rows,
                    sp_out, sn_out, sem):
    wid = lax.axis_index("s") * NC + lax.axis_index("c")
    base = wid * BPW

    pltpu.sync_copy(t_idx.at[pl.ds(base, BPW)], idx_t)
    pltpu.sync_copy(cp_idx.at[pl.ds(base, BPW)], idx_cp)
    pltpu.sync_copy(cn_idx.at[pl.ds(base * NNEG, BPW * NNEG)], idx_cn)

    lane = lax.iota(jnp.int32, L)

    def do_chunk(c):
        cps = []
        cps.append(pltpu.async_copy(
            t_tab.at[idx_t.at[pl.ds(c * CH, CH)]], t_rows, sem))
        cps.append(pltpu.async_copy(
            c_tab.at[idx_cp.at[pl.ds(c * CH, CH)]], cp_rows, sem))
        for j in range(N_CN_DMAS):
            cps.append(pltpu.async_copy(
                c_tab.at[idx_cn.at[pl.ds(c * CN_PER_CHUNK + j * CN_DMA, CN_DMA)]],
                cn_rows.at[pl.ds(j * CN_DMA, CN_DMA)], sem))
        for h in cps:
            h.wait()

        for g in range(CH // L):
            b0 = g * L
            row_t = b0 + lane
            row_cn0 = (b0 + lane) * NNEG

            def dot_step(d, carry):
                s_p, s_n = carry
                dv = jnp.full((L,), d, jnp.int32)
                td = plsc.load_gather(t_rows, [row_t, dv])
                cpd = plsc.load_gather(cp_rows, [row_t, dv])
                cs = plsc.load_gather(cn_rows, [row_cn0, dv])
                for n in range(1, NNEG):
                    cs = cs + plsc.load_gather(cn_rows, [row_cn0 + n, dv])
                return s_p + td * cpd, s_n + td * cs

            zero = jnp.zeros((L,), jnp.float32)
            s_p, s_n = lax.fori_loop(0, DIM, dot_step, (zero, zero))
            sp_out[pl.ds(c * CH + b0, L)] = -s_p
            sn_out[pl.ds(c * CH + b0, L)] = s_n

    def loop_body(c, _):
        do_chunk(c)
        return 0

    lax.fori_loop(0, NCH, loop_body, 0)

    pltpu.sync_copy(sp_out, out.at[pl.ds(base, BPW)])
    pltpu.sync_copy(sn_out, out.at[pl.ds(B + base, BPW)])


def _sc_scores(t_tab, c_tab, t_idx, cp_idx, cn_idx):
    mesh = plsc.VectorSubcoreMesh(core_axis_name="c", subcore_axis_name="s")
    return pl.kernel(
        _sc_scores_body,
        out_type=jax.ShapeDtypeStruct((2 * B,), jnp.float32),
        mesh=mesh,
        compiler_params=pltpu.CompilerParams(
            use_tc_tiling_on_sc=False, needs_layout_passes=False),
        scratch_types=[
            pltpu.VMEM((BPW,), jnp.int32),
            pltpu.VMEM((BPW,), jnp.int32),
            pltpu.VMEM((BPW * NNEG,), jnp.int32),
            pltpu.VMEM((CH, DIM), jnp.float32),
            pltpu.VMEM((CH, DIM), jnp.float32),
            pltpu.VMEM((CN_PER_CHUNK, DIM), jnp.float32),
            pltpu.VMEM((BPW,), jnp.float32),
            pltpu.VMEM((BPW,), jnp.float32),
            pltpu.SemaphoreType.DMA,
        ],
    )(t_tab, c_tab, t_idx, cp_idx, cn_idx)


def _loss_body(s_ref, o_ref):
    x = s_ref[...]
    z = jnp.minimum(x, 0.0) - jnp.log(1.0 + jnp.exp(-jnp.abs(x)))
    o_ref[0, 0] = jnp.sum(z) * (1.0 / B)


def _tc_loss(scores):
    out = pl.pallas_call(
        _loss_body,
        out_shape=jax.ShapeDtypeStruct((1, 1), jnp.float32),
        out_specs=pl.BlockSpec(memory_space=pltpu.SMEM),
    )(scores.reshape(128, 2 * B // 128))
    return out[0, 0]


@jax.jit
def kernel(t_vocab_embs, c_vocab_embs, t, cp, cn):
    t_i = t.astype(jnp.int32)
    cp_i = cp.astype(jnp.int32)
    cn_i = cn.astype(jnp.int32).reshape(B * NNEG)
    scores = _sc_scores(t_vocab_embs, c_vocab_embs, t_i, cp_i, cn_i)
    return _tc_loss(scores)


# speedup vs baseline: 4.4882x; 1.0304x over previous
import jax
import jax.numpy as jnp
from jax import lax
from jax.experimental import pallas as pl
from jax.experimental.pallas import tpu as pltpu
from jax.experimental.pallas import tpu_sc as plsc

VOCAB = 1000000
DIM = 64
PD = 128
B = 16384
NNEG = 20

NC = 2
NS = 16
L = 16
NW = NC * NS
BPW = B // NW
CH = 16
NCH = BPW // CH
CNC = CH * NNEG
CN_SPLIT = ((0, 128), (128, 128), (256, 64))


def _fire_chunk(c_tabs, idxs, bufs, sem, c):
    t_tab, c_tab = c_tabs
    idx_t, idx_cp, idx_cn = idxs
    t_rows, cp_rows, cn_rows = bufs
    pltpu.async_copy(t_tab.at[idx_t.at[pl.ds(c * CH, CH)]], t_rows, sem)
    pltpu.async_copy(c_tab.at[idx_cp.at[pl.ds(c * CH, CH)]], cp_rows, sem)
    for off, n in CN_SPLIT:
        pltpu.async_copy(c_tab.at[idx_cn.at[pl.ds(c * CNC + off, n)]],
                         cn_rows.at[pl.ds(off, n)], sem)


def _drain_chunk(c_tabs, idxs, bufs, sem, c):
    t_tab, c_tab = c_tabs
    idx_t, idx_cp, idx_cn = idxs
    t_rows, cp_rows, cn_rows = bufs
    pltpu.make_async_copy(t_tab.at[idx_t.at[pl.ds(c * CH, CH)]], t_rows, sem).wait()
    pltpu.make_async_copy(c_tab.at[idx_cp.at[pl.ds(c * CH, CH)]], cp_rows, sem).wait()
    for off, n in CN_SPLIT:
        pltpu.make_async_copy(c_tab.at[idx_cn.at[pl.ds(c * CNC + off, n)]],
                              cn_rows.at[pl.ds(off, n)], sem).wait()


def _sc_scores_body(t_tab, c_tab, t_idx, cp_idx, cn_idx, out,
                    idx_t, idx_cp, idx_cn, t_rows, cp_rows, cn_rows,
                    sp_out, sn_out, sem0, sem1):
    wid = lax.axis_index("s") * NC + lax.axis_index("c")
    base = wid * BPW

    pltpu.sync_copy(t_idx.at[pl.ds(base, BPW)], idx_t)
    pltpu.sync_copy(cp_idx.at[pl.ds(base, BPW)], idx_cp)
    pltpu.sync_copy(cn_idx.at[pl.ds(base * NNEG, BPW * NNEG)], idx_cn)

    tabs = (t_tab, c_tab)
    idxs = (idx_t, idx_cp, idx_cn)
    bufs = [(t_rows.at[k], cp_rows.at[k], cn_rows.at[k]) for k in (0, 1)]
    sems = (sem0, sem1)

    lane = lax.iota(jnp.int32, L)
    cn_row = [lane * NNEG + n for n in range(NNEG)]

    def compute(c, k):
        def dot_step(d, carry):
            s_p, s_n = carry
            dv = jnp.full((L,), d, jnp.int32)
            td = plsc.load_gather(t_rows.at[k], [lane, dv])
            cpd = plsc.load_gather(cp_rows.at[k], [lane, dv])
            cs0 = plsc.load_gather(cn_rows.at[k], [cn_row[0], dv])
            cs1 = plsc.load_gather(cn_rows.at[k], [cn_row[1], dv])
            for n in range(2, NNEG, 2):
                cs0 = cs0 + plsc.load_gather(cn_rows.at[k], [cn_row[n], dv])
                cs1 = cs1 + plsc.load_gather(cn_rows.at[k], [cn_row[n + 1], dv])
            return s_p + td * cpd, s_n + td * (cs0 + cs1)

        zero = jnp.zeros((L,), jnp.float32)
        s_p, s_n = lax.fori_loop(0, DIM, dot_step, (zero, zero), unroll=4)
        sp_out[pl.ds(c * CH, L)] = -s_p
        sn_out[pl.ds(c * CH, L)] = s_n

    _fire_chunk(tabs, idxs, bufs[0], sems[0], 0)

    def loop_body(i, _):
        c0 = 2 * i
        _fire_chunk(tabs, idxs, bufs[1], sems[1], c0 + 1)
        _drain_chunk(tabs, idxs, bufs[0], sems[0], c0)
        compute(c0, 0)

        @pl.when(c0 + 2 < NCH)
        def _():
            _fire_chunk(tabs, idxs, bufs[0], sems[0], c0 + 2)
        _drain_chunk(tabs, idxs, bufs[1], sems[1], c0 + 1)
        compute(c0 + 1, 1)
        return 0

    lax.fori_loop(0, NCH // 2, loop_body, 0)

    pltpu.sync_copy(sp_out, out.at[pl.ds(base, BPW)])
    pltpu.sync_copy(sn_out, out.at[pl.ds(B + base, BPW)])


def _sc_scores(t_tab, c_tab, t_idx, cp_idx, cn_idx):
    mesh = plsc.VectorSubcoreMesh(core_axis_name="c", subcore_axis_name="s")
    return pl.kernel(
        _sc_scores_body,
        out_type=jax.ShapeDtypeStruct((2 * B,), jnp.float32),
        mesh=mesh,
        compiler_params=pltpu.CompilerParams(needs_layout_passes=False),
        scratch_types=[
            pltpu.VMEM((BPW,), jnp.int32),
            pltpu.VMEM((BPW,), jnp.int32),
            pltpu.VMEM((BPW * NNEG,), jnp.int32),
            pltpu.VMEM((2, CH, PD), jnp.float32),
            pltpu.VMEM((2, CH, PD), jnp.float32),
            pltpu.VMEM((2, CNC, PD), jnp.float32),
            pltpu.VMEM((BPW,), jnp.float32),
            pltpu.VMEM((BPW,), jnp.float32),
            pltpu.SemaphoreType.DMA,
            pltpu.SemaphoreType.DMA,
        ],
    )(t_tab, c_tab, t_idx, cp_idx, cn_idx)


def _loss_body(s_ref, o_ref):
    x = s_ref[...]
    z = jnp.minimum(x, 0.0) - jnp.log(1.0 + jnp.exp(-jnp.abs(x)))
    o_ref[0, 0] = jnp.sum(z) * (1.0 / B)


def _tc_loss(scores):
    out = pl.pallas_call(
        _loss_body,
        out_shape=jax.ShapeDtypeStruct((1, 1), jnp.float32),
        out_specs=pl.BlockSpec(memory_space=pltpu.SMEM),
    )(scores.reshape(128, 2 * B // 128))
    return out[0, 0]


@jax.jit
def kernel(t_vocab_embs, c_vocab_embs, t, cp, cn):
    t_pad = jnp.pad(t_vocab_embs, ((0, 0), (0, PD - DIM)))
    c_pad = jnp.pad(c_vocab_embs, ((0, 0), (0, PD - DIM)))
    t_i = t.astype(jnp.int32)
    cp_i = cp.astype(jnp.int32)
    cn_i = cn.astype(jnp.int32).reshape(B * NNEG)
    scores = _sc_scores(t_pad, c_pad, t_i, cp_i, cn_i)
    return _tc_loss(scores)


# speedup vs baseline: 5.6429x; 1.2573x over previous
import jax
import jax.numpy as jnp
from jax import lax
from jax.experimental import pallas as pl
from jax.experimental.pallas import tpu as pltpu
from jax.experimental.pallas import tpu_sc as plsc

VOCAB = 1000000
DIM = 64
PD = 128
B = 16384
NNEG = 20

NC = 2
NS = 16
L = 16
NW = NC * NS
BPW = B // NW
CH = 16
NCH = BPW // CH
CNC = CH * NNEG
CN_SPLIT = ((0, 128), (128, 128), (256, 64))


def _fire_chunk(c_tabs, idxs, bufs, sem, c):
    t_tab, c_tab = c_tabs
    idx_t, idx_cp, idx_cn = idxs
    t_rows, cp_rows, cn_rows = bufs
    pltpu.async_copy(t_tab.at[idx_t.at[pl.ds(c * CH, CH)]], t_rows, sem)
    pltpu.async_copy(c_tab.at[idx_cp.at[pl.ds(c * CH, CH)]], cp_rows, sem)
    for off, n in CN_SPLIT:
        pltpu.async_copy(c_tab.at[idx_cn.at[pl.ds(c * CNC + off, n)]],
                         cn_rows.at[pl.ds(off, n)], sem)


def _drain_chunk(c_tabs, idxs, bufs, sem, c):
    t_tab, c_tab = c_tabs
    idx_t, idx_cp, idx_cn = idxs
    t_rows, cp_rows, cn_rows = bufs
    pltpu.make_async_copy(t_tab.at[idx_t.at[pl.ds(c * CH, CH)]], t_rows, sem).wait()
    pltpu.make_async_copy(c_tab.at[idx_cp.at[pl.ds(c * CH, CH)]], cp_rows, sem).wait()
    for off, n in CN_SPLIT:
        pltpu.make_async_copy(c_tab.at[idx_cn.at[pl.ds(c * CNC + off, n)]],
                              cn_rows.at[pl.ds(off, n)], sem).wait()


def _sc_scores_body(t_tab, c_tab, t_idx, cp_idx, cn_idx, out,
                    idx_t, idx_cp, idx_cn, t_rows, cp_rows, cn_rows,
                    sp_out, sn_out, sem0, sem1):
    wid = lax.axis_index("s") * NC + lax.axis_index("c")
    base = wid * BPW

    pltpu.sync_copy(t_idx.at[pl.ds(base, BPW)], idx_t)
    pltpu.sync_copy(cp_idx.at[pl.ds(base, BPW)], idx_cp)
    pltpu.sync_copy(cn_idx.at[pl.ds(base * NNEG, BPW * NNEG)], idx_cn)

    tabs = (t_tab, c_tab)
    idxs = (idx_t, idx_cp, idx_cn)
    bufs = [(t_rows.at[k], cp_rows.at[k], cn_rows.at[k]) for k in (0, 1)]
    sems = (sem0, sem1)

    lane = lax.iota(jnp.int32, L)
    cn_row = [lane * NNEG + n for n in range(NNEG)]

    def compute(c, k):
        def dot_step(d, carry):
            s_p, s_n = carry
            dv = (lane + d) & (DIM - 1)
            td = plsc.load_gather(t_rows.at[k], [lane, dv])
            cpd = plsc.load_gather(cp_rows.at[k], [lane, dv])
            cs0 = plsc.load_gather(cn_rows.at[k], [cn_row[0], dv])
            cs1 = plsc.load_gather(cn_rows.at[k], [cn_row[1], dv])
            for n in range(2, NNEG, 2):
                cs0 = cs0 + plsc.load_gather(cn_rows.at[k], [cn_row[n], dv])
                cs1 = cs1 + plsc.load_gather(cn_rows.at[k], [cn_row[n + 1], dv])
            return s_p + td * cpd, s_n + td * (cs0 + cs1)

        zero = jnp.zeros((L,), jnp.float32)
        s_p, s_n = lax.fori_loop(0, DIM, dot_step, (zero, zero), unroll=4)
        sp_out[pl.ds(c * CH, L)] = -s_p
        sn_out[pl.ds(c * CH, L)] = s_n

    _fire_chunk(tabs, idxs, bufs[0], sems[0], 0)

    def loop_body(i, _):
        c0 = 2 * i
        _fire_chunk(tabs, idxs, bufs[1], sems[1], c0 + 1)
        _drain_chunk(tabs, idxs, bufs[0], sems[0], c0)
        compute(c0, 0)

        @pl.when(c0 + 2 < NCH)
        def _():
            _fire_chunk(tabs, idxs, bufs[0], sems[0], c0 + 2)
        _drain_chunk(tabs, idxs, bufs[1], sems[1], c0 + 1)
        compute(c0 + 1, 1)
        return 0

    lax.fori_loop(0, NCH // 2, loop_body, 0)

    pltpu.sync_copy(sp_out, out.at[pl.ds(base, BPW)])
    pltpu.sync_copy(sn_out, out.at[pl.ds(B + base, BPW)])


def _sc_scores(t_tab, c_tab, t_idx, cp_idx, cn_idx):
    mesh = plsc.VectorSubcoreMesh(core_axis_name="c", subcore_axis_name="s")
    return pl.kernel(
        _sc_scores_body,
        out_type=jax.ShapeDtypeStruct((2 * B,), jnp.float32),
        mesh=mesh,
        compiler_params=pltpu.CompilerParams(needs_layout_passes=False),
        scratch_types=[
            pltpu.VMEM((BPW,), jnp.int32),
            pltpu.VMEM((BPW,), jnp.int32),
            pltpu.VMEM((BPW * NNEG,), jnp.int32),
            pltpu.VMEM((2, CH, PD), jnp.float32),
            pltpu.VMEM((2, CH, PD), jnp.float32),
            pltpu.VMEM((2, CNC, PD), jnp.float32),
            pltpu.VMEM((BPW,), jnp.float32),
            pltpu.VMEM((BPW,), jnp.float32),
            pltpu.SemaphoreType.DMA,
            pltpu.SemaphoreType.DMA,
        ],
    )(t_tab, c_tab, t_idx, cp_idx, cn_idx)


def _loss_body(s_ref, o_ref):
    x = s_ref[...]
    z = jnp.minimum(x, 0.0) - jnp.log(1.0 + jnp.exp(-jnp.abs(x)))
    o_ref[0, 0] = jnp.sum(z) * (1.0 / B)


def _tc_loss(scores):
    out = pl.pallas_call(
        _loss_body,
        out_shape=jax.ShapeDtypeStruct((1, 1), jnp.float32),
        out_specs=pl.BlockSpec(memory_space=pltpu.SMEM),
    )(scores.reshape(128, 2 * B // 128))
    return out[0, 0]


@jax.jit
def kernel(t_vocab_embs, c_vocab_embs, t, cp, cn):
    t_pad = jnp.pad(t_vocab_embs, ((0, 0), (0, PD - DIM)))
    c_pad = jnp.pad(c_vocab_embs, ((0, 0), (0, PD - DIM)))
    t_i = t.astype(jnp.int32)
    cp_i = cp.astype(jnp.int32)
    cn_i = cn.astype(jnp.int32).reshape(B * NNEG)
    scores = _sc_scores(t_pad, c_pad, t_i, cp_i, cn_i)
    return _tc_loss(scores)
